# Initial kernel scaffold; baseline (speedup 1.0000x reference)
import jax
import jax.numpy as jnp
from jax.experimental import pallas as pl


def kernel(x, edge_index, edge_type, weights, roots, biases):
    raise NotImplementedError("write your pallas kernel here")


# speedup vs baseline: 14.2562x; 14.2562x over previous
import functools

import jax
import jax.numpy as jnp
from jax import lax
from jax.experimental import pallas as pl
from jax.experimental.pallas import tpu as pltpu
from jax.experimental.pallas import tpu_sc as plsc

_N = 10000
_E = 320000
_D = 128
_R = 8
_L = 3

_NC = 2
_NS = 16
_NW = _NC * _NS
_KE = 128
_EPW = -(-(_E // _NW) // _KE) * _KE
_EPAD = _EPW * _NW
_RN = _R * _N
_RNP = _RN + _KE
_CH = _RNP // _NS
_EPT = _EPAD // _NS
_NBN = 10
_BN = _N // _NBN
_FCH = 40
_NFCH = _N // _FCH
_FIT = -(-_NFCH // _NS)

_sc_mesh = plsc.VectorSubcoreMesh(core_axis_name="c", subcore_axis_name="s")


@functools.partial(
    pl.kernel,
    out_type=jax.ShapeDtypeStruct((_EPAD,), jnp.float32),
    mesh=_sc_mesh,
    scratch_types=[
        pltpu.VMEM_SHARED((_RNP,), jnp.float32),
        pltpu.VMEM((_CH,), jnp.float32),
        pltpu.VMEM((_KE,), jnp.int32),
        pltpu.VMEM((_KE,), jnp.float32),
        pltpu.VMEM((_KE,), jnp.float32),
        pltpu.SemaphoreType.DMA,
    ],
)
def _scale_kernel(rdst_hbm, scale_hbm, cnt_sh, zv, eidx_v, val_v, ones_v, sem):
    sid = lax.axis_index("s")
    cid = lax.axis_index("c")
    wid = sid * _NC + cid
    z16 = jnp.zeros((16,), jnp.float32)
    o16 = jnp.ones((16,), jnp.float32)
    for j in range(_KE // 16):
        ones_v[pl.ds(j * 16, 16)] = o16

    for j in range(_CH // 16):
        zv[pl.ds(j * 16, 16)] = z16
    pltpu.sync_copy(zv, cnt_sh.at[pl.ds(sid * _CH, _CH)])
    plsc.subcore_barrier()

    def count_step(b, carry):
        base = sid * _EPT + b * _KE
        pltpu.sync_copy(rdst_hbm.at[pl.ds(base, _KE)], eidx_v)
        pltpu.sync_copy(ones_v, cnt_sh.at[eidx_v], add=True)
        return carry

    lax.fori_loop(0, _EPT // _KE, count_step, 0)
    plsc.subcore_barrier()

    pltpu.sync_copy(cnt_sh.at[pl.ds(sid * _CH, _CH)], zv)
    iota16 = lax.broadcasted_iota(jnp.int32, (16,), 0)
    for j in range(_CH // 16):
        c = zv[pl.ds(j * 16, 16)]
        slot = sid * _CH + j * 16 + iota16
        inv = 1.0 / jnp.maximum(c, 1.0)
        zv[pl.ds(j * 16, 16)] = jnp.where(slot < _RN, inv, 0.0)
    pltpu.sync_copy(zv, cnt_sh.at[pl.ds(sid * _CH, _CH)])
    plsc.subcore_barrier()

    def gather_step(b, carry):
        base = wid * _EPW + b * _KE
        pltpu.sync_copy(rdst_hbm.at[pl.ds(base, _KE)], eidx_v)
        pltpu.async_copy(cnt_sh.at[eidx_v], val_v, sem).wait()
        pltpu.sync_copy(val_v, scale_hbm.at[pl.ds(base, _KE)])
        return carry

    lax.fori_loop(0, _EPW // _KE, gather_step, 0)


@functools.partial(
    pl.kernel,
    out_type=jax.ShapeDtypeStruct((_NC, _N, _D), jnp.float32),
    mesh=_sc_mesh,
    scratch_types=[
        pltpu.VMEM_SHARED((_N, _D), jnp.float32),
        pltpu.VMEM((_KE, _D), jnp.float32),
        pltpu.VMEM((_FCH, _D), jnp.float32),
        pltpu.VMEM((_KE,), jnp.int32),
        pltpu.VMEM((_KE,), jnp.int32),
        pltpu.VMEM((_KE,), jnp.float32),
        pltpu.SemaphoreType.DMA,
    ],
)
def _agg_kernel(hflat_hbm, gidx_hbm, dst_hbm, scale_hbm, agg_hbm,
                acc_sh, rows_v, zr, gidx_v, dst_v, sval_v, sem):
    sid = lax.axis_index("s")
    cid = lax.axis_index("c")
    wid = sid * _NC + cid
    z16 = jnp.zeros((16,), jnp.float32)
    for rr in range(_FCH):
        for c in range(_D // 16):
            zr[rr, pl.ds(c * 16, 16)] = z16
    for k in range(_FIT):
        chunk = k * _NS + sid
        @pl.when(chunk < _NFCH)
        def _():
            r0 = pl.multiple_of(chunk * _FCH, 8)
            pltpu.sync_copy(zr, acc_sh.at[pl.ds(r0, _FCH), :])
    plsc.subcore_barrier()

    def step(b, carry):
        base = wid * _EPW + b * _KE
        pltpu.sync_copy(gidx_hbm.at[pl.ds(base, _KE)], gidx_v)
        pltpu.sync_copy(dst_hbm.at[pl.ds(base, _KE)], dst_v)
        pltpu.sync_copy(scale_hbm.at[pl.ds(base, _KE)], sval_v)
        pltpu.async_copy(hflat_hbm.at[gidx_v], rows_v, sem).wait()

        def scale_step(jj, c2):
            sv = sval_v[pl.ds(jj * 16, 16)]
            for u in range(16):
                j = jj * 16 + u
                s = sv[u]
                for c in range(_D // 16):
                    rows_v[j, pl.ds(c * 16, 16)] = rows_v[j, pl.ds(c * 16, 16)] * s
            return c2

        lax.fori_loop(0, _KE // 16, scale_step, 0)
        pltpu.sync_copy(rows_v, acc_sh.at[dst_v], add=True)
        return carry

    lax.fori_loop(0, _EPW // _KE, step, 0)
    plsc.subcore_barrier()

    for k in range(_FIT):
        chunk = k * _NS + sid
        @pl.when(chunk < _NFCH)
        def _():
            r0 = pl.multiple_of(chunk * _FCH, 8)
            pltpu.sync_copy(acc_sh.at[pl.ds(r0, _FCH), :], zr)
            pltpu.sync_copy(zr, agg_hbm.at[cid, pl.ds(r0, _FCH), :])


def _mm_body(h_ref, w_ref, out_ref):
    out_ref[0] = jnp.dot(h_ref[...], w_ref[0], preferred_element_type=jnp.float32)


def _matmul(h, w9):
    return pl.pallas_call(
        _mm_body,
        grid=(_R + 1, _NBN),
        in_specs=[
            pl.BlockSpec((_BN, _D), lambda r, i: (i, 0)),
            pl.BlockSpec((1, _D, _D), lambda r, i: (r, 0, 0)),
        ],
        out_specs=pl.BlockSpec((1, _BN, _D), lambda r, i: (r, i, 0)),
        out_shape=jax.ShapeDtypeStruct((_R + 1, _N, _D), jnp.float32),
    )(h, w9)


def _comb_body(h9_ref, agg_ref, b_ref, out_ref, *, relu):
    v = h9_ref[0] + agg_ref[0] + agg_ref[1] + b_ref[...]
    if relu:
        v = jnp.maximum(v, 0.0)
    out_ref[...] = v


def _combine(h9, agg, bias2d, relu):
    return pl.pallas_call(
        functools.partial(_comb_body, relu=relu),
        grid=(_NBN,),
        in_specs=[
            pl.BlockSpec((1, _BN, _D), lambda i: (_R, i, 0)),
            pl.BlockSpec((_NC, _BN, _D), lambda i: (0, i, 0)),
            pl.BlockSpec((1, _D), lambda i: (0, 0)),
        ],
        out_specs=pl.BlockSpec((_BN, _D), lambda i: (i, 0)),
        out_shape=jax.ShapeDtypeStruct((_N, _D), jnp.float32),
    )(h9, agg, bias2d)


def kernel(x, edge_index, edge_type, weights, roots, biases):
    src = edge_index[0].astype(jnp.int32)
    dst = edge_index[1].astype(jnp.int32)
    et = edge_type.astype(jnp.int32)
    gidx = et * _N + src
    rdst = et * _N + dst
    pad = _EPAD - _E
    gidx_p = jnp.pad(gidx, (0, pad))
    dst_p = jnp.pad(dst, (0, pad))
    rdst_p = jnp.pad(rdst, (0, pad), constant_values=_RN)
    w9 = jnp.concatenate([weights, roots[:, None]], axis=1)

    scale = _scale_kernel(rdst_p)
    h = x
    for l in range(_L):
        h9 = _matmul(h, w9[l])
        agg = _agg_kernel(h9.reshape(((_R + 1) * _N, _D)), gidx_p, dst_p, scale)
        h = _combine(h9, agg, biases[l][None], relu=(l < _L - 1))
    return h
